# baseline (speedup 1.0000x reference)
import jax
import jax.numpy as jnp
from jax import lax
from jax.experimental import pallas as pl
from jax.experimental.pallas import tpu as pltpu
from jax.experimental.pallas import tpu_sc as plsc

B = 16384
F = 26
V = 100000
D = 16

NW = 32
BF = B * F
CHUNK = 128
ROWS_PER_W = BF // NW // CHUNK
LCM_VECS = 208 // 16


def _body(x_hbm, tab_hbm, out_hbm, idx_v, pat_v, rows_v, gsem):
    cid = lax.axis_index("c")
    sid = lax.axis_index("s")
    wid = sid * 2 + cid
    row0 = wid * ROWS_PER_W

    pltpu.sync_copy(x_hbm.at[pl.ds(row0, ROWS_PER_W)], idx_v)

    for p in range(LCM_VECS):
        vals = lax.iota(jnp.int32, 16) + (p * 16)
        pat_v[p, :] = lax.rem(vals, F) * V

    def chunk_body(j, carry):
        for c in range(8):
            pv = lax.rem(j * 8 + c, LCM_VECS)
            idx_v[j, pl.ds(c * 16, 16)] = (
                idx_v[j, pl.ds(c * 16, 16)] + pat_v[pv, :]
            )
        pltpu.async_copy(tab_hbm.at[idx_v.at[j]], rows_v, gsem).wait()
        pltpu.sync_copy(rows_v, out_hbm.at[pl.ds((row0 + j) * CHUNK, CHUNK)])
        return carry

    lax.fori_loop(0, ROWS_PER_W, chunk_body, 0)


def kernel(X, tables):
    x32 = X.astype(jnp.int32).reshape(BF // CHUNK, CHUNK)
    tab = tables.reshape(F * V, D)
    mesh = plsc.VectorSubcoreMesh(core_axis_name="c", subcore_axis_name="s")
    out = pl.kernel(
        _body,
        mesh=mesh,
        out_type=jax.ShapeDtypeStruct((BF, D), jnp.float32),
        scratch_types=[
            pltpu.VMEM((ROWS_PER_W, CHUNK), jnp.int32),
            pltpu.VMEM((LCM_VECS, 16), jnp.int32),
            pltpu.VMEM((CHUNK, D), jnp.float32),
            pltpu.SemaphoreType.DMA,
        ],
        compiler_params=pltpu.CompilerParams(use_tc_tiling_on_sc=False),
    )(x32, tab)
    return out.reshape(B, F, D)


# speedup vs baseline: 1.0491x; 1.0491x over previous
import jax
import jax.numpy as jnp
from jax import lax
from jax.experimental import pallas as pl
from jax.experimental.pallas import tpu as pltpu
from jax.experimental.pallas import tpu_sc as plsc

B = 16384
F = 26
V = 100000
D = 16

NW = 32
BF = B * F
CHUNK = 128
ROWS_PER_W = BF // NW // CHUNK
LCM_VECS = 208 // 16
G = 8
NG = ROWS_PER_W // G
GROWS = G * CHUNK


def _body(x_hbm, tab_hbm, out_hbm, idx_v, pat_v, rows_v, gsem, osem):
    cid = lax.axis_index("c")
    sid = lax.axis_index("s")
    wid = sid * 2 + cid
    row0 = wid * ROWS_PER_W
    out0 = row0 * CHUNK

    pltpu.sync_copy(x_hbm.at[pl.ds(row0, ROWS_PER_W)], idx_v)

    for p in range(LCM_VECS):
        pat_v[p, :] = lax.rem(lax.iota(jnp.int32, 16) + p * 16, F) * V

    def add_offsets(g):
        for k in range(G):
            j = g * G + k
            for c in range(8):
                pv = lax.rem(j * 8 + c, LCM_VECS)
                idx_v[j, pl.ds(c * 16, 16)] = (
                    idx_v[j, pl.ds(c * 16, 16)] + pat_v[pv, :]
                )

    def fire_group(g):
        boff = lax.rem(g, 2) * GROWS
        for k in range(G):
            j = g * G + k
            pltpu.async_copy(
                tab_hbm.at[idx_v.at[j]],
                rows_v.at[pl.ds(boff + k * CHUNK, CHUNK)],
                gsem,
            )

    def wait_group():
        pltpu.make_async_copy(
            tab_hbm.at[pl.ds(0, GROWS)], rows_v.at[pl.ds(0, GROWS)], gsem
        ).wait()

    def fire_out(g):
        boff = lax.rem(g, 2) * GROWS
        pltpu.async_copy(
            rows_v.at[pl.ds(boff, GROWS)],
            out_hbm.at[pl.ds(out0 + g * GROWS, GROWS)],
            osem,
        )

    def wait_out():
        pltpu.make_async_copy(
            tab_hbm.at[pl.ds(0, GROWS)], rows_v.at[pl.ds(0, GROWS)], osem
        ).wait()

    add_offsets(jnp.int32(0))
    fire_group(jnp.int32(0))

    def group_body(g, carry):
        @pl.when(g < NG - 1)
        def _():
            add_offsets(g + 1)

        @pl.when(g >= 1)
        def _():
            wait_out()

        @pl.when(g < NG - 1)
        def _():
            fire_group(g + 1)

        wait_group()
        fire_out(g)
        return carry

    lax.fori_loop(0, NG, group_body, 0)
    wait_out()


def kernel(X, tables):
    x32 = X.astype(jnp.int32).reshape(BF // CHUNK, CHUNK)
    tab = tables.reshape(F * V, D)
    mesh = plsc.VectorSubcoreMesh(core_axis_name="c", subcore_axis_name="s")
    out = pl.kernel(
        _body,
        mesh=mesh,
        out_type=jax.ShapeDtypeStruct((BF, D), jnp.float32),
        scratch_types=[
            pltpu.VMEM((ROWS_PER_W, CHUNK), jnp.int32),
            pltpu.VMEM((LCM_VECS, 16), jnp.int32),
            pltpu.VMEM((2 * GROWS, D), jnp.float32),
            pltpu.SemaphoreType.DMA,
            pltpu.SemaphoreType.DMA,
        ],
        compiler_params=pltpu.CompilerParams(use_tc_tiling_on_sc=False),
    )(x32, tab)
    return out.reshape(B, F, D)


# speedup vs baseline: 1.3688x; 1.3047x over previous
import jax
import jax.numpy as jnp
from jax import lax
from jax.experimental import pallas as pl
from jax.experimental.pallas import tpu as pltpu
from jax.experimental.pallas import tpu_sc as plsc

B = 16384
F = 26
V = 100000
D = 16

NW = 32
BPW = B // NW
CHUNK = 128
KPW = BPW // CHUNK
NBLK = F * KPW
NB = 4
DEPTH = 3


def _body(x_hbm, tab_hbm, op_hbm, xv, fidx, gbuf, tbuf, gsem, osem):
    cid = lax.axis_index("c")
    sid = lax.axis_index("s")
    wid = sid * 2 + cid
    b0 = wid * BPW

    pltpu.sync_copy(x_hbm.at[pl.ds(b0 * F, BPW * F)], xv)

    iota = lax.iota(jnp.int32, 16)
    zeros = jnp.zeros((16,), jnp.int32)
    iota_f = iota * F

    def build_and_fire(blk):
        f = lax.shift_right_logical(blk, 2)
        k = lax.bitwise_and(blk, 3)
        slot = lax.bitwise_and(blk, NB - 1)
        fv = f * V
        for q in range(8):
            addrs = iota_f + ((k * CHUNK + q * 16) * F + f)
            fidx[slot, pl.ds(q * 16, 16)] = plsc.load_gather(xv, [addrs]) + fv
        pltpu.async_copy(tab_hbm.at[fidx.at[slot]], gbuf.at[slot], gsem)

    def drain_and_emit(blk):
        f = lax.shift_right_logical(blk, 2)
        k = lax.bitwise_and(blk, 3)
        slot = lax.bitwise_and(blk, NB - 1)
        bt = wid * KPW + k
        pltpu.make_async_copy(
            tab_hbm.at[pl.ds(0, CHUNK)], gbuf.at[slot], gsem
        ).wait()
        @pl.when(blk >= NB)
        def _():
            pltpu.make_async_copy(
                op_hbm.at[0, 0, 0], tbuf.at[slot, pl.ds(0, 8)], osem
            ).wait()
            pltpu.make_async_copy(
                op_hbm.at[0, 0, 0], tbuf.at[slot, pl.ds(8, 8)], osem
            ).wait()

        def tr(j, carry):
            v = gbuf[slot, j, :]
            plsc.store_scatter(tbuf.at[slot], [iota, zeros + j], v)
            return carry

        lax.fori_loop(0, CHUNK, tr, 0)
        pltpu.async_copy(tbuf.at[slot, pl.ds(0, 8)], op_hbm.at[f, 0, bt], osem)
        pltpu.async_copy(tbuf.at[slot, pl.ds(8, 8)], op_hbm.at[f, 1, bt], osem)

    for blk in range(DEPTH):
        build_and_fire(jnp.int32(blk))

    def loop_body(blk, carry):
        build_and_fire(blk)
        drain_and_emit(blk - DEPTH)
        return carry

    lax.fori_loop(DEPTH, NBLK, loop_body, 0)
    for r in range(DEPTH):
        drain_and_emit(jnp.int32(NBLK - DEPTH + r))
    for r in range(NB):
        pltpu.make_async_copy(
            op_hbm.at[0, 0, 0], tbuf.at[r, pl.ds(0, 8)], osem
        ).wait()
        pltpu.make_async_copy(
            op_hbm.at[0, 0, 0], tbuf.at[r, pl.ds(8, 8)], osem
        ).wait()


def kernel(X, tables):
    mesh = plsc.VectorSubcoreMesh(core_axis_name="c", subcore_axis_name="s")
    op = pl.kernel(
        _body,
        mesh=mesh,
        out_type=jax.ShapeDtypeStruct((F, 2, B // CHUNK, 8, CHUNK), jnp.float32),
        scratch_types=[
            pltpu.VMEM((BPW * F,), jnp.int32),
            pltpu.VMEM((NB, CHUNK), jnp.int32),
            pltpu.VMEM((NB, CHUNK, D), jnp.float32),
            pltpu.VMEM((NB, D, CHUNK), jnp.float32),
            pltpu.SemaphoreType.DMA,
            pltpu.SemaphoreType.DMA,
        ],
        compiler_params=pltpu.CompilerParams(
            use_tc_tiling_on_sc=False, needs_layout_passes=False
        ),
    )(X.astype(jnp.int32).reshape(B * F), tables.reshape(F * V, D))
    return (
        op.transpose(0, 1, 3, 2, 4)
        .reshape(F, D, B)
        .transpose(2, 0, 1)
    )
